# baseline (speedup 1.0000x reference)
import functools

import jax
import jax.numpy as jnp
from jax import lax
from jax.experimental import pallas as pl
from jax.experimental.pallas import tpu as pltpu
from jax.experimental.pallas import tpu_sc as plsc

_MIN_NORM = 1e-15
_BALL_EPS = 4e-3
_N, _E, _D = 10000, 320000, 128

_NC, _NS = 2, 16
_N_PAD = 10240
_ROWS_PER_TILE = _N_PAD // _NS
_CHUNK = 80
_E_PER_CORE = _E // _NC
_E_PER_TILE = _E_PER_CORE // _NS
_NPT = _E_PER_TILE // _CHUNK
_ZROWS = 80
_DEG_W = 16


def _rnorm(v):
    return jnp.maximum(jnp.sqrt(jnp.sum(v * v, axis=-1, keepdims=True)), _MIN_NORM)


def _artanh(v):
    v = jnp.clip(v, -1.0 + 1e-7, 1.0 - 1e-7)
    return 0.5 * (jnp.log1p(v) - jnp.log1p(-v))


def _proj(v):
    n = _rnorm(v)
    maxnorm = 1.0 - _BALL_EPS
    return jnp.where(n > maxnorm, v / n * maxnorm, v)


def _expmap0(v):
    n = _rnorm(v)
    return jnp.tanh(n) * v / n


def _logmap0(v):
    n = _rnorm(v)
    return v / n * _artanh(n)


def _phase_a_body(x_ref, w_ref, b_ref, o_ref):
    x = x_ref[...]
    w = w_ref[...]
    b = b_ref[...]

    x_hyp = _proj(_expmap0(x))

    xn = _rnorm(x_hyp)
    mx = lax.dot_general(x_hyp, w, (((1,), (1,)), ((), ())),
                         preferred_element_type=jnp.float32,
                         precision=lax.Precision.HIGHEST)
    mxn = _rnorm(mx)
    res = jnp.tanh(mxn / xn * _artanh(xn)) * mx / mxn
    res = jnp.where(jnp.all(mx == 0.0, axis=-1, keepdims=True), 0.0, res)
    res = _proj(res)

    hb = _proj(_expmap0(b))

    x2 = jnp.sum(res * res, axis=-1, keepdims=True)
    y2 = jnp.sum(hb * hb, axis=-1, keepdims=True)
    xy = jnp.sum(res * hb, axis=-1, keepdims=True)
    num = (1.0 + 2.0 * xy + y2) * res + (1.0 - x2) * hb
    den = jnp.maximum(1.0 + 2.0 * xy + x2 * y2, _MIN_NORM)
    ma = _proj(num / den)

    o_ref[...] = _logmap0(ma).astype(jnp.bfloat16)


def _phase_a(x, W, b2):
    blk = 1000
    return pl.pallas_call(
        _phase_a_body,
        grid=(_N // blk,),
        in_specs=[
            pl.BlockSpec((blk, _D), lambda i: (i, 0)),
            pl.BlockSpec((_D, _D), lambda i: (0, 0)),
            pl.BlockSpec((1, _D), lambda i: (0, 0)),
        ],
        out_specs=pl.BlockSpec((blk, _D), lambda i: (i, 0)),
        out_shape=jax.ShapeDtypeStruct((_N, _D), jnp.bfloat16),
    )(x, W, b2)


def _sc_agg(h_tan, edge_index):
    @functools.partial(
        pl.kernel,
        out_type=[
            jax.ShapeDtypeStruct((_NC, _N_PAD, _D), jnp.bfloat16),
            jax.ShapeDtypeStruct((_NC, _N_PAD, _DEG_W), jnp.float32),
        ],
        mesh=plsc.VectorSubcoreMesh(core_axis_name="c", subcore_axis_name="s"),
        compiler_params=pltpu.CompilerParams(use_tc_tiling_on_sc=False,
                                             needs_layout_passes=False),
        scratch_types=[
            pltpu.VMEM_SHARED((_N_PAD, _D), jnp.bfloat16),
            pltpu.VMEM_SHARED((_NS, _N_PAD), jnp.float32),
            pltpu.VMEM((2, _CHUNK), jnp.int32),
            pltpu.VMEM((2, _CHUNK), jnp.int32),
            pltpu.VMEM((_CHUNK, _D), jnp.bfloat16),
            pltpu.VMEM((_CHUNK, _D), jnp.bfloat16),
            pltpu.VMEM((_N_PAD,), jnp.float32),
            pltpu.VMEM((_NS, _ROWS_PER_TILE), jnp.float32),
            pltpu.VMEM((_ROWS_PER_TILE, _DEG_W), jnp.float32),
            pltpu.SemaphoreType.DMA,
            pltpu.SemaphoreType.DMA,
            pltpu.SemaphoreType.DMA,
            pltpu.SemaphoreType.DMA,
        ],
    )
    def sc_kernel(h_hbm, ei_hbm, acc_out, deg_out,
                  acc_sp, hist_sp, idx0, idx1, rows0, rows1, hist, hsum, degc,
                  gs0, gs1, ss0, ss1):
        c = lax.axis_index("c")
        s = lax.axis_index("s")
        idx = (idx0, idx1)
        rows = (rows0, rows1)
        gs, ss = (gs0, gs1), (ss0, ss1)

        zfb = jnp.zeros((32,), jnp.bfloat16)
        zf = jnp.zeros((16,), jnp.float32)
        onesv = jnp.ones((16,), jnp.float32)

        def zbody(i, _):
            for j in range(_D // 32):
                rows0[i, pl.ds(j * 32, 32)] = zfb
            for j in range(_N_PAD // _ZROWS // 16):
                hist[pl.ds((i * (_N_PAD // _ZROWS // 16) + j) * 16, 16)] = zf
            return 0

        lax.fori_loop(0, _ZROWS, zbody, 0)

        for k in range(_ROWS_PER_TILE // _ZROWS):
            off = s * _ROWS_PER_TILE + k * _ZROWS
            pltpu.sync_copy(rows0, acc_sp.at[pl.ds(off, _ZROWS)])
        plsc.subcore_barrier()

        tile_base = c * _E_PER_CORE + s * _E_PER_TILE

        def gather_wait(k):
            pltpu.make_async_copy(h_hbm.at[idx[k].at[0]], rows[k], gs[k]).wait()

        def scatter_start(k):
            pltpu.async_copy(rows[k], acc_sp.at[idx[k].at[1]], ss[k], add=True)

        def scatter_wait(k):
            pltpu.make_async_copy(rows[k], acc_sp.at[idx[k].at[1]], ss[k]).wait()

        def step(i, cur, prv):
            @pl.when(i >= 1)
            def _():
                gather_wait(prv)
                scatter_start(prv)

            @pl.when(i >= 2)
            def _():
                scatter_wait(cur)

            base = tile_base + i * _CHUNK
            pltpu.sync_copy(ei_hbm.at[pl.ds(0, 2), pl.ds(base, _CHUNK)],
                            idx[cur])
            pltpu.async_copy(h_hbm.at[idx[cur].at[0]], rows[cur], gs[cur])

            for g in range(_CHUNK // 16):
                d = idx[cur][1, pl.ds(16 * g, 16)]
                plsc.addupdate_scatter(hist, [d], onesv)

        def body(i, _):
            @pl.when(i % 2 == 0)
            def _():
                step(i, 0, 1)

            @pl.when(i % 2 == 1)
            def _():
                step(i, 1, 0)
            return 0

        lax.fori_loop(0, _NPT, body, 0)

        last = (_NPT - 1) % 2
        scatter_wait(1 - last)
        gather_wait(last)
        scatter_start(last)
        scatter_wait(last)

        pltpu.sync_copy(hist, hist_sp.at[s])
        plsc.subcore_barrier()

        off = s * _ROWS_PER_TILE
        pltpu.sync_copy(acc_sp.at[pl.ds(off, _ROWS_PER_TILE)],
                        acc_out.at[c, pl.ds(off, _ROWS_PER_TILE)])

        pltpu.sync_copy(hist_sp.at[pl.ds(0, _NS), pl.ds(off, _ROWS_PER_TILE)],
                        hsum)
        lane0 = jnp.zeros((16,), jnp.int32)

        def dbody(g, _):
            acc16 = hsum[0, pl.ds(16 * g, 16)]
            for q in range(1, _NS):
                acc16 = acc16 + hsum[q, pl.ds(16 * g, 16)]
            rid = lax.iota(jnp.int32, 16) + 16 * g
            plsc.store_scatter(degc, [rid, lane0], acc16)
            return 0

        lax.fori_loop(0, _ROWS_PER_TILE // 16, dbody, 0)
        pltpu.sync_copy(degc, deg_out.at[c, pl.ds(off, _ROWS_PER_TILE)])

    return sc_kernel(h_tan, edge_index)


def _phase_c_body(a0_ref, a1_ref, d0_ref, d1_ref, o_ref):
    agg = a0_ref[0].astype(jnp.float32) + a1_ref[0].astype(jnp.float32)
    deg = d0_ref[0][:, :1] + d1_ref[0][:, :1]
    agg = agg / jnp.maximum(deg, 1.0)
    out = _proj(_expmap0(agg))
    xt = jnp.maximum(_logmap0(out), 0.0)
    o_ref[...] = _proj(_expmap0(xt))


def _phase_c(acc, deg):
    blk = 1000
    return pl.pallas_call(
        _phase_c_body,
        grid=(_N // blk,),
        in_specs=[
            pl.BlockSpec((1, blk, _D), lambda i: (0, i, 0)),
            pl.BlockSpec((1, blk, _D), lambda i: (1, i, 0)),
            pl.BlockSpec((1, blk, _DEG_W), lambda i: (0, i, 0)),
            pl.BlockSpec((1, blk, _DEG_W), lambda i: (1, i, 0)),
        ],
        out_specs=pl.BlockSpec((blk, _D), lambda i: (i, 0)),
        out_shape=jax.ShapeDtypeStruct((_N, _D), jnp.float32),
    )(acc, acc, deg, deg)


def kernel(x, edge_index, W, b):
    h_tan = _phase_a(x, W, b.reshape(1, -1))
    acc, deg = _sc_agg(h_tan, edge_index)
    return _phase_c(acc, deg)


# speedup vs baseline: 1.2412x; 1.2412x over previous
import functools

import jax
import jax.numpy as jnp
from jax import lax
from jax.experimental import pallas as pl
from jax.experimental.pallas import tpu as pltpu
from jax.experimental.pallas import tpu_sc as plsc

_MIN_NORM = 1e-15
_BALL_EPS = 4e-3
_N, _E, _D = 10000, 320000, 128

_NC, _NS = 2, 16
_N_PAD = 10240
_ROWS_PER_TILE = _N_PAD // _NS
_CHUNK = 80
_BLK = 5
_E_PER_CORE = _E // _NC
_E_PER_TILE = _E_PER_CORE // _NS
_NPT = _E_PER_TILE // _CHUNK
_NBLK = _NPT // _BLK
_ZROWS = 80
_DEG_W = 16


def _rnorm(v):
    return jnp.maximum(jnp.sqrt(jnp.sum(v * v, axis=-1, keepdims=True)), _MIN_NORM)


def _artanh(v):
    v = jnp.clip(v, -1.0 + 1e-7, 1.0 - 1e-7)
    return 0.5 * (jnp.log1p(v) - jnp.log1p(-v))


def _proj(v):
    n = _rnorm(v)
    maxnorm = 1.0 - _BALL_EPS
    return jnp.where(n > maxnorm, v / n * maxnorm, v)


def _expmap0(v):
    n = _rnorm(v)
    return jnp.tanh(n) * v / n


def _logmap0(v):
    n = _rnorm(v)
    return v / n * _artanh(n)


def _phase_a_body(x_ref, w_ref, b_ref, o_ref):
    x = x_ref[...]
    w = w_ref[...]
    b = b_ref[...]

    x_hyp = _proj(_expmap0(x))

    xn = _rnorm(x_hyp)
    mx = lax.dot_general(x_hyp, w, (((1,), (1,)), ((), ())),
                         preferred_element_type=jnp.float32,
                         precision=lax.Precision.HIGHEST)
    mxn = _rnorm(mx)
    res = jnp.tanh(mxn / xn * _artanh(xn)) * mx / mxn
    res = jnp.where(jnp.all(mx == 0.0, axis=-1, keepdims=True), 0.0, res)
    res = _proj(res)

    hb = _proj(_expmap0(b))

    x2 = jnp.sum(res * res, axis=-1, keepdims=True)
    y2 = jnp.sum(hb * hb, axis=-1, keepdims=True)
    xy = jnp.sum(res * hb, axis=-1, keepdims=True)
    num = (1.0 + 2.0 * xy + y2) * res + (1.0 - x2) * hb
    den = jnp.maximum(1.0 + 2.0 * xy + x2 * y2, _MIN_NORM)
    ma = _proj(num / den)

    o_ref[...] = _logmap0(ma).astype(jnp.bfloat16)


def _phase_a(x, W, b2):
    blk = 1000
    return pl.pallas_call(
        _phase_a_body,
        grid=(_N // blk,),
        in_specs=[
            pl.BlockSpec((blk, _D), lambda i: (i, 0)),
            pl.BlockSpec((_D, _D), lambda i: (0, 0)),
            pl.BlockSpec((1, _D), lambda i: (0, 0)),
        ],
        out_specs=pl.BlockSpec((blk, _D), lambda i: (i, 0)),
        out_shape=jax.ShapeDtypeStruct((_N, _D), jnp.bfloat16),
    )(x, W, b2)


def _sc_agg(h_tan, edge_index):
    @functools.partial(
        pl.kernel,
        out_type=[
            jax.ShapeDtypeStruct((_NC, _N_PAD, _D), jnp.bfloat16),
            jax.ShapeDtypeStruct((_NC, _N_PAD, _DEG_W), jnp.float32),
        ],
        mesh=plsc.VectorSubcoreMesh(core_axis_name="c", subcore_axis_name="s"),
        compiler_params=pltpu.CompilerParams(use_tc_tiling_on_sc=False,
                                             needs_layout_passes=False),
        scratch_types=[
            pltpu.VMEM_SHARED((_N_PAD, _D), jnp.bfloat16),
            pltpu.VMEM_SHARED((_NS, _N_PAD), jnp.float32),
            pltpu.VMEM((2, _BLK, _CHUNK), jnp.int32),
            pltpu.VMEM((2, _BLK, _CHUNK), jnp.int32),
            pltpu.VMEM((_CHUNK, _D), jnp.bfloat16),
            pltpu.VMEM((_CHUNK, _D), jnp.bfloat16),
            pltpu.VMEM((_N_PAD,), jnp.float32),
            pltpu.VMEM((_NS, _ROWS_PER_TILE), jnp.float32),
            pltpu.VMEM((_ROWS_PER_TILE, _DEG_W), jnp.float32),
            pltpu.SemaphoreType.DMA,
            pltpu.SemaphoreType.DMA,
            pltpu.SemaphoreType.DMA,
            pltpu.SemaphoreType.DMA,
            pltpu.SemaphoreType.DMA,
            pltpu.SemaphoreType.DMA,
        ],
    )
    def sc_kernel(h_hbm, ei_hbm, acc_out, deg_out,
                  acc_sp, hist_sp, ib0, ib1, rows0, rows1, hist, hsum, degc,
                  is0, is1, gs0, gs1, ss0, ss1):
        c = lax.axis_index("c")
        s = lax.axis_index("s")
        ib = (ib0, ib1)
        rows = (rows0, rows1)
        isem, gs, ss = (is0, is1), (gs0, gs1), (ss0, ss1)

        zfb = jnp.zeros((32,), jnp.bfloat16)
        zf = jnp.zeros((16,), jnp.float32)
        onesv = jnp.ones((16,), jnp.float32)

        def zbody(i, _):
            for j in range(_D // 32):
                rows0[i, pl.ds(j * 32, 32)] = zfb
            for j in range(_N_PAD // _ZROWS // 16):
                hist[pl.ds((i * (_N_PAD // _ZROWS // 16) + j) * 16, 16)] = zf
            return 0

        lax.fori_loop(0, _ZROWS, zbody, 0)

        for k in range(_ROWS_PER_TILE // _ZROWS):
            off = s * _ROWS_PER_TILE + k * _ZROWS
            pltpu.sync_copy(rows0, acc_sp.at[pl.ds(off, _ZROWS)])
        plsc.subcore_barrier()

        chunk0 = (c * _NS + s) * _NPT

        def idx_start(b, p):
            pltpu.async_copy(
                ei_hbm.at[pl.ds(0, 2), pl.ds(chunk0 + b * _BLK, _BLK)],
                ib[p], isem[p])

        def idx_wait(b, p):
            pltpu.make_async_copy(
                ei_hbm.at[pl.ds(0, 2), pl.ds(chunk0 + b * _BLK, _BLK)],
                ib[p], isem[p]).wait()

        def gather_start(p, k, r):
            pltpu.async_copy(h_hbm.at[ib[p].at[0, k]], rows[r], gs[r])

        def gather_wait(p, k, r):
            pltpu.make_async_copy(h_hbm.at[ib[p].at[0, k]], rows[r],
                                  gs[r]).wait()

        def scatter_start(p, k, r):
            pltpu.async_copy(rows[r], acc_sp.at[ib[p].at[1, k]], ss[r],
                             add=True)

        def scatter_wait(p, k, r):
            pltpu.make_async_copy(rows[r], acc_sp.at[ib[p].at[1, k]],
                                  ss[r]).wait()

        def block_body(b, p, first):
            for k in range(_BLK):
                r = (p + k) % 2
                if not (first and k == 0):
                    pp, pk = (p, k - 1) if k >= 1 else (1 - p, _BLK - 1)
                    gather_wait(pp, pk, 1 - r)
                    scatter_start(pp, pk, 1 - r)
                if not (first and k <= 1):
                    qp, qk = (p, k - 2) if k >= 2 else (1 - p, _BLK - 2 + k)
                    scatter_wait(qp, qk, r)
                if k == 2 and not first:
                    @pl.when(b < _NBLK - 1)
                    def _():
                        idx_start(b + 1, 1 - p)
                gather_start(p, k, r)
                for g in range(_CHUNK // 16):
                    d = ib[p][1, k, pl.ds(16 * g, 16)]
                    plsc.addupdate_scatter(hist, [d], onesv)

        pltpu.sync_copy(ei_hbm.at[pl.ds(0, 2), pl.ds(chunk0, _BLK)], ib[0])
        idx_start(1, 1)
        block_body(0, 0, True)

        def body_dispatch(b, _):
            @pl.when(b % 2 == 1)
            def _():
                idx_wait(b, 1)
                block_body(b, 1, False)

            @pl.when(b % 2 == 0)
            def _():
                idx_wait(b, 0)
                block_body(b, 0, False)
            return 0

        lax.fori_loop(1, _NBLK, body_dispatch, 0)

        lp = (_NBLK - 1) % 2
        scatter_wait(lp, _BLK - 2, (lp + _BLK - 2) % 2)
        gather_wait(lp, _BLK - 1, (lp + _BLK - 1) % 2)
        scatter_start(lp, _BLK - 1, (lp + _BLK - 1) % 2)
        scatter_wait(lp, _BLK - 1, (lp + _BLK - 1) % 2)

        pltpu.sync_copy(hist, hist_sp.at[s])
        plsc.subcore_barrier()

        off = s * _ROWS_PER_TILE
        pltpu.sync_copy(acc_sp.at[pl.ds(off, _ROWS_PER_TILE)],
                        acc_out.at[c, pl.ds(off, _ROWS_PER_TILE)])

        pltpu.sync_copy(hist_sp.at[pl.ds(0, _NS), pl.ds(off, _ROWS_PER_TILE)],
                        hsum)
        lane0 = jnp.zeros((16,), jnp.int32)

        def dbody(g, _):
            acc16 = hsum[0, pl.ds(16 * g, 16)]
            for q in range(1, _NS):
                acc16 = acc16 + hsum[q, pl.ds(16 * g, 16)]
            rid = lax.iota(jnp.int32, 16) + 16 * g
            plsc.store_scatter(degc, [rid, lane0], acc16)
            return 0

        lax.fori_loop(0, _ROWS_PER_TILE // 16, dbody, 0)
        pltpu.sync_copy(degc, deg_out.at[c, pl.ds(off, _ROWS_PER_TILE)])

    return sc_kernel(h_tan, edge_index)


def _phase_c_body(a0_ref, a1_ref, d0_ref, d1_ref, o_ref):
    agg = a0_ref[0].astype(jnp.float32) + a1_ref[0].astype(jnp.float32)
    deg = d0_ref[0][:, :1] + d1_ref[0][:, :1]
    agg = agg / jnp.maximum(deg, 1.0)
    out = _proj(_expmap0(agg))
    xt = jnp.maximum(_logmap0(out), 0.0)
    o_ref[...] = _proj(_expmap0(xt))


def _phase_c(acc, deg):
    blk = 1000
    return pl.pallas_call(
        _phase_c_body,
        grid=(_N // blk,),
        in_specs=[
            pl.BlockSpec((1, blk, _D), lambda i: (0, i, 0)),
            pl.BlockSpec((1, blk, _D), lambda i: (1, i, 0)),
            pl.BlockSpec((1, blk, _DEG_W), lambda i: (0, i, 0)),
            pl.BlockSpec((1, blk, _DEG_W), lambda i: (1, i, 0)),
        ],
        out_specs=pl.BlockSpec((blk, _D), lambda i: (i, 0)),
        out_shape=jax.ShapeDtypeStruct((_N, _D), jnp.float32),
    )(acc, acc, deg, deg)


def kernel(x, edge_index, W, b):
    h_tan = _phase_a(x, W, b.reshape(1, -1))
    acc, deg = _sc_agg(h_tan, edge_index.reshape(2, _E // _CHUNK, _CHUNK))
    return _phase_c(acc, deg)


# speedup vs baseline: 1.5316x; 1.2340x over previous
import functools

import jax
import jax.numpy as jnp
from jax import lax
from jax.experimental import pallas as pl
from jax.experimental.pallas import tpu as pltpu
from jax.experimental.pallas import tpu_sc as plsc

_MIN_NORM = 1e-15
_BALL_EPS = 4e-3
_N, _E, _D = 10000, 320000, 128

_NC, _NS = 2, 16
_N_PAD = 10240
_ROWS_PER_TILE = _N_PAD // _NS
_CHUNK = 80
_BLK = 5
_NR = 5
_E_PER_CORE = _E // _NC
_E_PER_TILE = _E_PER_CORE // _NS
_NPT = _E_PER_TILE // _CHUNK
_NBLK = _NPT // _BLK
_ZROWS = 80
_DEG_W = 16


def _rnorm(v):
    return jnp.maximum(jnp.sqrt(jnp.sum(v * v, axis=-1, keepdims=True)), _MIN_NORM)


def _artanh(v):
    v = jnp.clip(v, -1.0 + 1e-7, 1.0 - 1e-7)
    return 0.5 * (jnp.log1p(v) - jnp.log1p(-v))


def _proj(v):
    n = _rnorm(v)
    maxnorm = 1.0 - _BALL_EPS
    return jnp.where(n > maxnorm, v / n * maxnorm, v)


def _expmap0(v):
    n = _rnorm(v)
    return jnp.tanh(n) * v / n


def _logmap0(v):
    n = _rnorm(v)
    return v / n * _artanh(n)


def _phase_a_body(x_ref, w_ref, b_ref, o_ref):
    x = x_ref[...]
    w = w_ref[...]
    b = b_ref[...]

    x_hyp = _proj(_expmap0(x))

    xn = _rnorm(x_hyp)
    mx = lax.dot_general(x_hyp, w, (((1,), (1,)), ((), ())),
                         preferred_element_type=jnp.float32,
                         precision=lax.Precision.HIGHEST)
    mxn = _rnorm(mx)
    res = jnp.tanh(mxn / xn * _artanh(xn)) * mx / mxn
    res = jnp.where(jnp.all(mx == 0.0, axis=-1, keepdims=True), 0.0, res)
    res = _proj(res)

    hb = _proj(_expmap0(b))

    x2 = jnp.sum(res * res, axis=-1, keepdims=True)
    y2 = jnp.sum(hb * hb, axis=-1, keepdims=True)
    xy = jnp.sum(res * hb, axis=-1, keepdims=True)
    num = (1.0 + 2.0 * xy + y2) * res + (1.0 - x2) * hb
    den = jnp.maximum(1.0 + 2.0 * xy + x2 * y2, _MIN_NORM)
    ma = _proj(num / den)

    o_ref[...] = _logmap0(ma).astype(jnp.bfloat16)


def _phase_a(x, W, b2):
    blk = 1000
    return pl.pallas_call(
        _phase_a_body,
        grid=(_N // blk,),
        in_specs=[
            pl.BlockSpec((blk, _D), lambda i: (i, 0)),
            pl.BlockSpec((_D, _D), lambda i: (0, 0)),
            pl.BlockSpec((1, _D), lambda i: (0, 0)),
        ],
        out_specs=pl.BlockSpec((blk, _D), lambda i: (i, 0)),
        out_shape=jax.ShapeDtypeStruct((_N, _D), jnp.bfloat16),
    )(x, W, b2)


def _sc_agg(h_tan, edge_index):
    @functools.partial(
        pl.kernel,
        out_type=[
            jax.ShapeDtypeStruct((_NC, _N_PAD, _D), jnp.bfloat16),
            jax.ShapeDtypeStruct((_NC, _N_PAD, _DEG_W), jnp.float32),
        ],
        mesh=plsc.VectorSubcoreMesh(core_axis_name="c", subcore_axis_name="s"),
        compiler_params=pltpu.CompilerParams(use_tc_tiling_on_sc=False,
                                             needs_layout_passes=False),
        scratch_types=[
            pltpu.VMEM_SHARED((_N_PAD, _D), jnp.bfloat16),
            pltpu.VMEM_SHARED((_NS, _N_PAD), jnp.float32),
            pltpu.VMEM((2, _BLK, _CHUNK), jnp.int32),
            pltpu.VMEM((2, _BLK, _CHUNK), jnp.int32),
            pltpu.VMEM((_NR, _CHUNK, _D), jnp.bfloat16),
            pltpu.VMEM((_N_PAD,), jnp.float32),
            pltpu.VMEM((_NS, _ROWS_PER_TILE), jnp.float32),
            pltpu.VMEM((_ROWS_PER_TILE, _DEG_W), jnp.float32),
            pltpu.SemaphoreType.DMA,
            pltpu.SemaphoreType.DMA,
            pltpu.SemaphoreType.DMA,
            pltpu.SemaphoreType.DMA,
            pltpu.SemaphoreType.DMA,
            pltpu.SemaphoreType.DMA,
            pltpu.SemaphoreType.DMA,
            pltpu.SemaphoreType.DMA,
            pltpu.SemaphoreType.DMA,
            pltpu.SemaphoreType.DMA,
            pltpu.SemaphoreType.DMA,
            pltpu.SemaphoreType.DMA,
        ],
    )
    def sc_kernel(h_hbm, ei_hbm, acc_out, deg_out,
                  acc_sp, hist_sp, ib0, ib1, rowsb, hist, hsum, degc,
                  is0, is1, g0, g1, g2, g3, g4, s0, s1, s2, s3, s4):
        c = lax.axis_index("c")
        s = lax.axis_index("s")
        ib = (ib0, ib1)
        isem = (is0, is1)
        gs = (g0, g1, g2, g3, g4)
        ss = (s0, s1, s2, s3, s4)

        zfb = jnp.zeros((32,), jnp.bfloat16)
        zf = jnp.zeros((16,), jnp.float32)
        onesv = jnp.ones((16,), jnp.float32)

        def zbody(i, _):
            for j in range(_D // 32):
                rowsb[0, i, pl.ds(j * 32, 32)] = zfb
            for j in range(_N_PAD // _ZROWS // 16):
                hist[pl.ds((i * (_N_PAD // _ZROWS // 16) + j) * 16, 16)] = zf
            return 0

        lax.fori_loop(0, _ZROWS, zbody, 0)

        for k in range(_ROWS_PER_TILE // _ZROWS):
            off = s * _ROWS_PER_TILE + k * _ZROWS
            pltpu.sync_copy(rowsb.at[0], acc_sp.at[pl.ds(off, _ZROWS)])
        plsc.subcore_barrier()

        chunk0 = (c * _NS + s) * _NPT

        def idx_start(b, p):
            pltpu.async_copy(
                ei_hbm.at[pl.ds(0, 2), pl.ds(chunk0 + b * _BLK, _BLK)],
                ib[p], isem[p])

        def idx_wait(b, p):
            pltpu.make_async_copy(
                ei_hbm.at[pl.ds(0, 2), pl.ds(chunk0 + b * _BLK, _BLK)],
                ib[p], isem[p]).wait()

        def _refs(pos):
            p = (pos // _BLK) % 2
            k = pos % _BLK
            r = pos % _NR
            return p, k, r

        def gather_start(pos):
            p, k, r = _refs(pos)
            pltpu.async_copy(h_hbm.at[ib[p].at[0, k]], rowsb.at[r], gs[r])

        def gather_wait(pos):
            p, k, r = _refs(pos)
            pltpu.make_async_copy(h_hbm.at[ib[p].at[0, k]], rowsb.at[r],
                                  gs[r]).wait()

        def scatter_start(pos):
            p, k, r = _refs(pos)
            pltpu.async_copy(rowsb.at[r], acc_sp.at[ib[p].at[1, k]], ss[r],
                             add=True)

        def scatter_wait(pos):
            p, k, r = _refs(pos)
            pltpu.make_async_copy(rowsb.at[r], acc_sp.at[ib[p].at[1, k]],
                                  ss[r]).wait()

        def histo(pos):
            p, k, _ = _refs(pos)
            for g in range(_CHUNK // 16):
                d = ib[p][1, k, pl.ds(16 * g, 16)]
                plsc.addupdate_scatter(hist, [d], onesv)

        pltpu.sync_copy(ei_hbm.at[pl.ds(0, 2), pl.ds(chunk0, _BLK)], ib[0])
        idx_start(1, 1)
        gather_start(0)
        gather_start(1)
        for j in range(_BLK):
            gather_wait(j)
            scatter_start(j)
            if j >= 2:
                scatter_wait(j - 2)
            if j == 3:
                idx_wait(1, 1)
            gather_start(j + 2)
            histo(j)

        def pair_body(u, _):
            for j in range(2 * _BLK):
                pos = _BLK + j
                gather_wait(pos)
                scatter_start(pos)
                scatter_wait(pos - 2)
                if j == 1:
                    idx_start(2 * u + 2, 0)
                if j == 3:
                    idx_wait(2 * u + 2, 0)
                if j == 6:
                    @pl.when(u < _NBLK // 2 - 1)
                    def _():
                        idx_start(2 * u + 3, 1)
                if j == 8:
                    @pl.when(u < _NBLK // 2 - 1)
                    def _():
                        idx_wait(2 * u + 3, 1)
                if j < 8:
                    gather_start(pos + 2)
                else:
                    @pl.when(u < _NBLK // 2 - 1)
                    def _():
                        gather_start(pos + 2)
                histo(pos)
            return 0

        lax.fori_loop(0, _NBLK // 2, pair_body, 0)

        scatter_wait(_BLK + 8)
        scatter_wait(_BLK + 9)

        pltpu.sync_copy(hist, hist_sp.at[s])
        plsc.subcore_barrier()

        off = s * _ROWS_PER_TILE
        pltpu.sync_copy(acc_sp.at[pl.ds(off, _ROWS_PER_TILE)],
                        acc_out.at[c, pl.ds(off, _ROWS_PER_TILE)])

        pltpu.sync_copy(hist_sp.at[pl.ds(0, _NS), pl.ds(off, _ROWS_PER_TILE)],
                        hsum)
        lane0 = jnp.zeros((16,), jnp.int32)

        def dbody(g, _):
            acc16 = hsum[0, pl.ds(16 * g, 16)]
            for q in range(1, _NS):
                acc16 = acc16 + hsum[q, pl.ds(16 * g, 16)]
            rid = lax.iota(jnp.int32, 16) + 16 * g
            plsc.store_scatter(degc, [rid, lane0], acc16)
            return 0

        lax.fori_loop(0, _ROWS_PER_TILE // 16, dbody, 0)
        pltpu.sync_copy(degc, deg_out.at[c, pl.ds(off, _ROWS_PER_TILE)])

    return sc_kernel(h_tan, edge_index)


def _phase_c_body(a0_ref, a1_ref, d0_ref, d1_ref, o_ref):
    agg = a0_ref[0].astype(jnp.float32) + a1_ref[0].astype(jnp.float32)
    deg = d0_ref[0][:, :1] + d1_ref[0][:, :1]
    agg = agg / jnp.maximum(deg, 1.0)
    out = _proj(_expmap0(agg))
    xt = jnp.maximum(_logmap0(out), 0.0)
    o_ref[...] = _proj(_expmap0(xt))


def _phase_c(acc, deg):
    blk = 1000
    return pl.pallas_call(
        _phase_c_body,
        grid=(_N // blk,),
        in_specs=[
            pl.BlockSpec((1, blk, _D), lambda i: (0, i, 0)),
            pl.BlockSpec((1, blk, _D), lambda i: (1, i, 0)),
            pl.BlockSpec((1, blk, _DEG_W), lambda i: (0, i, 0)),
            pl.BlockSpec((1, blk, _DEG_W), lambda i: (1, i, 0)),
        ],
        out_specs=pl.BlockSpec((blk, _D), lambda i: (i, 0)),
        out_shape=jax.ShapeDtypeStruct((_N, _D), jnp.float32),
    )(acc, acc, deg, deg)


def kernel(x, edge_index, W, b):
    h_tan = _phase_a(x, W, b.reshape(1, -1))
    acc, deg = _sc_agg(h_tan, edge_index.reshape(2, _E // _CHUNK, _CHUNK))
    return _phase_c(acc, deg)
